# initial kernel scaffold (unmeasured)
import functools

import jax
import jax.numpy as jnp
from jax import lax
from jax.experimental import pallas as pl
from jax.experimental.pallas import tpu as pltpu

B, SQ, H, D = 4, 32, 8, 128
SKV_SHARD = 4096
C = 512
NKV = SKV_SHARD // C
SCALE = D ** -0.5


def kernel(Q, K, V):
    def body(q_ref, k_ref, v_ref, out_ref,
             num_acc, den_acc, recv_num, recv_den,
             send_sems, recv_sems):
        h = pl.program_id(0)
        j = pl.program_id(1)
        my_x = lax.axis_index("x")
        my_y = lax.axis_index("y")
        my_z = lax.axis_index("z")
        partner = (my_x, 1 - my_y, my_z)

        @pl.when((h == 0) & (j == 0))
        def _():
            barrier_sem = pltpu.get_barrier_semaphore()
            pl.semaphore_signal(
                barrier_sem, inc=1,
                device_id=partner, device_id_type=pl.DeviceIdType.MESH,
            )
            pl.semaphore_wait(barrier_sem, 1)

        @pl.when(j == 0)
        def _():
            num_acc[h] = jnp.zeros((B, SQ, D), jnp.float32)
            den_acc[h] = jnp.zeros((B, SQ), jnp.float32)

        q = q_ref[:, :, 0, :]
        k = k_ref[:, :, 0, :]
        v = v_ref[:, :, 0, :]
        s = lax.dot_general(
            q, k, (((2,), (2,)), ((0,), (0,))),
            preferred_element_type=jnp.float32,
        ) * SCALE
        e = jnp.exp(s)
        den_acc[h] += jnp.sum(e, axis=-1)
        num_acc[h] += lax.dot_general(
            e, v, (((2,), (1,)), ((0,), (0,))),
            preferred_element_type=jnp.float32,
        )

        @pl.when((h == H - 1) & (j == NKV - 1))
        def _():
            rdma_num = pltpu.make_async_remote_copy(
                src_ref=num_acc, dst_ref=recv_num,
                send_sem=send_sems.at[0], recv_sem=recv_sems.at[0],
                device_id=partner, device_id_type=pl.DeviceIdType.MESH,
            )
            rdma_den = pltpu.make_async_remote_copy(
                src_ref=den_acc, dst_ref=recv_den,
                send_sem=send_sems.at[1], recv_sem=recv_sems.at[1],
                device_id=partner, device_id_type=pl.DeviceIdType.MESH,
            )
            rdma_num.start()
            rdma_den.start()
            rdma_num.wait()
            rdma_den.wait()

            total_num = num_acc[:] + recv_num[:]
            total_den = den_acc[:] + recv_den[:]
            o = total_num / total_den[..., None]
            out_ref[:] = jnp.transpose(o, (1, 2, 0, 3))

            @functools.partial(
                pl.run_scoped, exit_sem=pltpu.SemaphoreType.REGULAR
            )
            def _(exit_sem):
                pl.semaphore_signal(
                    exit_sem, inc=1,
                    device_id=partner, device_id_type=pl.DeviceIdType.MESH,
                )
                pl.semaphore_wait(exit_sem, 1)

    grid = (H, NKV)
    return pl.pallas_call(
        body,
        grid=grid,
        in_specs=[
            pl.BlockSpec((B, SQ, 1, D), lambda h, j: (0, 0, h, 0),
                         memory_space=pltpu.VMEM),
            pl.BlockSpec((B, C, 1, D), lambda h, j: (0, j, h, 0),
                         memory_space=pltpu.VMEM),
            pl.BlockSpec((B, C, 1, D), lambda h, j: (0, j, h, 0),
                         memory_space=pltpu.VMEM),
        ],
        out_specs=pl.BlockSpec((B, SQ, H, D), lambda h, j: (0, 0, 0, 0),
                               memory_space=pltpu.VMEM),
        out_shape=jax.ShapeDtypeStruct((B, SQ, H, D), jnp.float32),
        scratch_shapes=[
            pltpu.VMEM((H, B, SQ, D), jnp.float32),
            pltpu.VMEM((H, B, SQ), jnp.float32),
            pltpu.VMEM((H, B, SQ, D), jnp.float32),
            pltpu.VMEM((H, B, SQ), jnp.float32),
            pltpu.SemaphoreType.DMA((2,)),
            pltpu.SemaphoreType.DMA((2,)),
        ],
        compiler_params=pltpu.CompilerParams(collective_id=0),
    )(Q, K, V)


# baseline (device time: 110111 ns/iter reference)
import functools

import jax
import jax.numpy as jnp
from jax import lax
from jax.experimental import pallas as pl
from jax.experimental.pallas import tpu as pltpu

B, SQ, H, D = 4, 32, 8, 128
SKV_SHARD = 4096
C = 256
NKV = SKV_SHARD // C
SCALE = D ** -0.5


def kernel(Q, K, V):
    def body(q_ref, k_ref, v_ref, out_ref,
             num_acc, den_acc, recv_num, recv_den,
             send_sems, recv_sems):
        j = pl.program_id(0)
        my_x = lax.axis_index("x")
        my_y = lax.axis_index("y")
        my_z = lax.axis_index("z")
        partner = (my_x, 1 - my_y, my_z)

        @pl.when(j == 0)
        def _():
            barrier_sem = pltpu.get_barrier_semaphore()
            pl.semaphore_signal(
                barrier_sem, inc=1,
                device_id=partner, device_id_type=pl.DeviceIdType.MESH,
            )
            pl.semaphore_wait(barrier_sem, 1)

        for h in range(H):
            q = q_ref[:, :, h, :]
            k = k_ref[:, :, h, :]
            v = v_ref[:, :, h, :]
            s = lax.dot_general(
                q, k, (((2,), (2,)), ((0,), (0,))),
                preferred_element_type=jnp.float32,
            ) * SCALE
            e = jnp.exp(s)
            den_c = jnp.sum(e, axis=-1)
            num_c = lax.dot_general(
                e, v, (((2,), (1,)), ((0,), (0,))),
                preferred_element_type=jnp.float32,
            )

            @pl.when(j == 0)
            def _():
                num_acc[h] = num_c
                den_acc[h] = den_c

            @pl.when(j != 0)
            def _():
                num_acc[h] += num_c
                den_acc[h] += den_c

        @pl.when(j == NKV - 1)
        def _():
            rdma_num = pltpu.make_async_remote_copy(
                src_ref=num_acc, dst_ref=recv_num,
                send_sem=send_sems.at[0], recv_sem=recv_sems.at[0],
                device_id=partner, device_id_type=pl.DeviceIdType.MESH,
            )
            rdma_den = pltpu.make_async_remote_copy(
                src_ref=den_acc, dst_ref=recv_den,
                send_sem=send_sems.at[1], recv_sem=recv_sems.at[1],
                device_id=partner, device_id_type=pl.DeviceIdType.MESH,
            )
            rdma_num.start()
            rdma_den.start()
            rdma_num.wait()
            rdma_den.wait()

            total_num = num_acc[:] + recv_num[:]
            total_den = den_acc[:] + recv_den[:]
            o = total_num / total_den[..., None]
            out_ref[:] = jnp.transpose(o, (1, 2, 0, 3))

            @functools.partial(
                pl.run_scoped, exit_sem=pltpu.SemaphoreType.REGULAR
            )
            def _(exit_sem):
                pl.semaphore_signal(
                    exit_sem, inc=1,
                    device_id=partner, device_id_type=pl.DeviceIdType.MESH,
                )
                pl.semaphore_wait(exit_sem, 1)

    grid = (NKV,)
    return pl.pallas_call(
        body,
        grid=grid,
        in_specs=[
            pl.BlockSpec((B, SQ, H, D), lambda j: (0, 0, 0, 0),
                         memory_space=pltpu.VMEM),
            pl.BlockSpec((B, C, H, D), lambda j: (0, j, 0, 0),
                         memory_space=pltpu.VMEM),
            pl.BlockSpec((B, C, H, D), lambda j: (0, j, 0, 0),
                         memory_space=pltpu.VMEM),
        ],
        out_specs=pl.BlockSpec((B, SQ, H, D), lambda j: (0, 0, 0, 0),
                               memory_space=pltpu.VMEM),
        out_shape=jax.ShapeDtypeStruct((B, SQ, H, D), jnp.float32),
        scratch_shapes=[
            pltpu.VMEM((H, B, SQ, D), jnp.float32),
            pltpu.VMEM((H, B, SQ), jnp.float32),
            pltpu.VMEM((H, B, SQ, D), jnp.float32),
            pltpu.VMEM((H, B, SQ), jnp.float32),
            pltpu.SemaphoreType.DMA((2,)),
            pltpu.SemaphoreType.DMA((2,)),
        ],
        compiler_params=pltpu.CompilerParams(collective_id=0),
    )(Q, K, V)


# device time: 71348 ns/iter; 1.5433x vs baseline; 1.5433x over previous
import functools

import jax
import jax.numpy as jnp
from jax import lax
from jax.experimental import pallas as pl
from jax.experimental.pallas import tpu as pltpu

B, SQ, H, D = 4, 32, 8, 128
SKV_SHARD = 4096
SPLIT = 4
SKV_LOCAL = SKV_SHARD // SPLIT
SCALE = D ** -0.5

PARTS = [(0, 3), (3, 6), (6, 8)]
ORDER = [(0, 1, 2), (1, 2, 0), (2, 0, 1)]


def kernel(Q, K, V):
    def body(q_ref, k_hbm, v_hbm, out_ref,
             kv_buf, vv_buf, num_acc, den_acc, recv_num, recv_den,
             load_sems, send_sems, recv_sems, dsend_sems, drecv_sems):
        my_x = lax.axis_index("x")
        my_y = lax.axis_index("y")
        my_z = lax.axis_index("z")

        def nbr(axis):
            if axis == 0:
                return (1 - my_x, my_y, my_z)
            if axis == 1:
                return (my_x, 1 - my_y, my_z)
            return (my_x, my_y, 1 - my_z)

        neighbors = [nbr(0), nbr(1), nbr(2)]

        barrier_sem = pltpu.get_barrier_semaphore()
        for n in neighbors:
            pl.semaphore_signal(
                barrier_sem, inc=1,
                device_id=n, device_id_type=pl.DeviceIdType.MESH,
            )
        pl.semaphore_wait(barrier_sem, 3)

        base = (2 * my_x + my_z) * SKV_LOCAL
        k_load = pltpu.make_async_copy(
            k_hbm.at[:, pl.ds(base, SKV_LOCAL), :, :], kv_buf,
            load_sems.at[0],
        )
        v_load = pltpu.make_async_copy(
            v_hbm.at[:, pl.ds(base, SKV_LOCAL), :, :], vv_buf,
            load_sems.at[1],
        )
        k_load.start()
        v_load.start()
        k_load.wait()
        v_load.wait()

        for h in range(H):
            q = q_ref[:, :, h, :].astype(jnp.bfloat16)
            k = kv_buf[:, :, h, :].astype(jnp.bfloat16)
            v = vv_buf[:, :, h, :].astype(jnp.bfloat16)
            s = lax.dot_general(
                q, k, (((2,), (2,)), ((0,), (0,))),
                preferred_element_type=jnp.float32,
            ) * SCALE
            e = jnp.exp(s)
            den_acc[h] = jnp.sum(e, axis=-1)
            num_acc[h] = lax.dot_general(
                e.astype(jnp.bfloat16), v, (((2,), (1,)), ((0,), (0,))),
                preferred_element_type=jnp.float32,
            )

        for s in range(3):
            rdmas = []
            for p, (h0, h1) in enumerate(PARTS):
                target = nbr(ORDER[p][s])
                r_num = pltpu.make_async_remote_copy(
                    src_ref=num_acc.at[h0:h1],
                    dst_ref=recv_num.at[s, h0:h1],
                    send_sem=send_sems.at[s, p],
                    recv_sem=recv_sems.at[s, p],
                    device_id=target, device_id_type=pl.DeviceIdType.MESH,
                )
                r_den = pltpu.make_async_remote_copy(
                    src_ref=den_acc.at[h0:h1],
                    dst_ref=recv_den.at[s, h0:h1],
                    send_sem=dsend_sems.at[s, p],
                    recv_sem=drecv_sems.at[s, p],
                    device_id=target, device_id_type=pl.DeviceIdType.MESH,
                )
                r_num.start()
                r_den.start()
                rdmas += [r_num, r_den]
            for r in rdmas:
                r.wait()
            num_acc[:] += recv_num[s]
            den_acc[:] += recv_den[s]

        o = num_acc[:] / den_acc[:][..., None]
        out_ref[:] = jnp.transpose(o, (1, 2, 0, 3))

        @functools.partial(
            pl.run_scoped, exit_sem=pltpu.SemaphoreType.REGULAR
        )
        def _(exit_sem):
            for n in neighbors:
                pl.semaphore_signal(
                    exit_sem, inc=1,
                    device_id=n, device_id_type=pl.DeviceIdType.MESH,
                )
            pl.semaphore_wait(exit_sem, 3)

    return pl.pallas_call(
        body,
        in_specs=[
            pl.BlockSpec(memory_space=pltpu.VMEM),
            pl.BlockSpec(memory_space=pl.ANY),
            pl.BlockSpec(memory_space=pl.ANY),
        ],
        out_specs=pl.BlockSpec(memory_space=pltpu.VMEM),
        out_shape=jax.ShapeDtypeStruct((B, SQ, H, D), jnp.float32),
        scratch_shapes=[
            pltpu.VMEM((B, SKV_LOCAL, H, D), jnp.float32),
            pltpu.VMEM((B, SKV_LOCAL, H, D), jnp.float32),
            pltpu.VMEM((H, B, SQ, D), jnp.float32),
            pltpu.VMEM((H, B, SQ), jnp.float32),
            pltpu.VMEM((3, H, B, SQ, D), jnp.float32),
            pltpu.VMEM((3, H, B, SQ), jnp.float32),
            pltpu.SemaphoreType.DMA((2,)),
            pltpu.SemaphoreType.DMA((3, 3)),
            pltpu.SemaphoreType.DMA((3, 3)),
            pltpu.SemaphoreType.DMA((3, 3)),
            pltpu.SemaphoreType.DMA((3, 3)),
        ],
        compiler_params=pltpu.CompilerParams(
            collective_id=0, vmem_limit_bytes=64 * 1024 * 1024,
        ),
    )(Q, K, V)


# device time: 36156 ns/iter; 3.0454x vs baseline; 1.9733x over previous
import functools
import os

import jax
import jax.numpy as jnp
from jax import lax
from jax.experimental import pallas as pl
from jax.experimental.pallas import tpu as pltpu

B, SQ, H, D = 4, 32, 8, 128
SKV_SHARD = 4096
SPLIT = 4
SKV_LOCAL = SKV_SHARD // SPLIT
SCALE = D ** -0.5

PARTS = [(0, 3), (3, 6), (6, 8)]
ORDER = [(0, 1, 2), (1, 2, 0), (2, 0, 1)]

SKIP_AR = os.environ.get("SKIP_AR") == "1"
SKIP_COMPUTE = os.environ.get("SKIP_COMPUTE") == "1"


def kernel(Q, K, V):
    def body(q_ref, k_hbm, v_hbm, out_ref,
             kv_buf, vv_buf, num_acc, den_acc, recv_num, recv_den,
             load_sems, send_sems, recv_sems, dsend_sems, drecv_sems):
        my_x = lax.axis_index("x")
        my_y = lax.axis_index("y")
        my_z = lax.axis_index("z")

        def nbr(axis):
            if axis == 0:
                return (1 - my_x, my_y, my_z)
            if axis == 1:
                return (my_x, 1 - my_y, my_z)
            return (my_x, my_y, 1 - my_z)

        neighbors = [nbr(0), nbr(1), nbr(2)]

        barrier_sem = pltpu.get_barrier_semaphore()
        for n in neighbors:
            pl.semaphore_signal(
                barrier_sem, inc=1,
                device_id=n, device_id_type=pl.DeviceIdType.MESH,
            )
        pl.semaphore_wait(barrier_sem, 3)

        with jax.named_scope("kv_load"):
            base = (2 * my_x + my_z) * SKV_LOCAL
            k_load = pltpu.make_async_copy(
                k_hbm.at[:, pl.ds(base, SKV_LOCAL), :, :], kv_buf,
                load_sems.at[0],
            )
            v_load = pltpu.make_async_copy(
                v_hbm.at[:, pl.ds(base, SKV_LOCAL), :, :], vv_buf,
                load_sems.at[1],
            )
            k_load.start()
            v_load.start()
            k_load.wait()
            v_load.wait()

        for h in range(H if not SKIP_COMPUTE else 1):
            with jax.named_scope(f"head#h={h}"):
                q = q_ref[:, :, h, :].astype(jnp.bfloat16)
                k = kv_buf[:, :, h, :].astype(jnp.bfloat16)
                v = vv_buf[:, :, h, :].astype(jnp.bfloat16)
                s = lax.dot_general(
                    q, k, (((2,), (2,)), ((0,), (0,))),
                    preferred_element_type=jnp.float32,
                ) * SCALE
                e = jnp.exp(s)
                den_acc[h] = jnp.sum(e, axis=-1)
                num_acc[h] = lax.dot_general(
                    e.astype(jnp.bfloat16), v, (((2,), (1,)), ((0,), (0,))),
                    preferred_element_type=jnp.float32,
                )

        for s in range(3 if not SKIP_AR else 0):
            with jax.named_scope(f"ar_send#s={s}"):
                rdmas = []
                for p, (h0, h1) in enumerate(PARTS):
                    target = nbr(ORDER[p][s])
                    r_num = pltpu.make_async_remote_copy(
                        src_ref=num_acc.at[h0:h1],
                        dst_ref=recv_num.at[s, h0:h1],
                        send_sem=send_sems.at[s, p],
                        recv_sem=recv_sems.at[s, p],
                        device_id=target, device_id_type=pl.DeviceIdType.MESH,
                    )
                    r_den = pltpu.make_async_remote_copy(
                        src_ref=den_acc.at[h0:h1],
                        dst_ref=recv_den.at[s, h0:h1],
                        send_sem=dsend_sems.at[s, p],
                        recv_sem=drecv_sems.at[s, p],
                        device_id=target, device_id_type=pl.DeviceIdType.MESH,
                    )
                    r_num.start()
                    r_den.start()
                    rdmas += [r_num, r_den]
            with jax.named_scope(f"ar_wait#s={s}"):
                for r in rdmas:
                    r.wait()
            with jax.named_scope(f"ar_add#s={s}"):
                num_acc[:] += recv_num[s]
                den_acc[:] += recv_den[s]

        with jax.named_scope("epilogue"):
            o = num_acc[:] / den_acc[:][..., None]
            out_ref[:] = jnp.transpose(o, (1, 2, 0, 3))

        @functools.partial(
            pl.run_scoped, exit_sem=pltpu.SemaphoreType.REGULAR
        )
        def _(exit_sem):
            for n in neighbors:
                pl.semaphore_signal(
                    exit_sem, inc=1,
                    device_id=n, device_id_type=pl.DeviceIdType.MESH,
                )
            pl.semaphore_wait(exit_sem, 3)

    return pl.pallas_call(
        body,
        in_specs=[
            pl.BlockSpec(memory_space=pltpu.VMEM),
            pl.BlockSpec(memory_space=pl.ANY),
            pl.BlockSpec(memory_space=pl.ANY),
        ],
        out_specs=pl.BlockSpec(memory_space=pltpu.VMEM),
        out_shape=jax.ShapeDtypeStruct((B, SQ, H, D), jnp.float32),
        scratch_shapes=[
            pltpu.VMEM((B, SKV_LOCAL, H, D), jnp.float32),
            pltpu.VMEM((B, SKV_LOCAL, H, D), jnp.float32),
            pltpu.VMEM((H, B, SQ, D), jnp.float32),
            pltpu.VMEM((H, B, SQ), jnp.float32),
            pltpu.VMEM((3, H, B, SQ, D), jnp.float32),
            pltpu.VMEM((3, H, B, SQ), jnp.float32),
            pltpu.SemaphoreType.DMA((2,)),
            pltpu.SemaphoreType.DMA((3, 3)),
            pltpu.SemaphoreType.DMA((3, 3)),
            pltpu.SemaphoreType.DMA((3, 3)),
            pltpu.SemaphoreType.DMA((3, 3)),
        ],
        compiler_params=pltpu.CompilerParams(
            collective_id=0, vmem_limit_bytes=64 * 1024 * 1024,
        ),
    )(Q, K, V)


# device time: 32577 ns/iter; 3.3800x vs baseline; 1.1099x over previous
import functools
import os

import jax
import jax.numpy as jnp
from jax import lax
from jax.experimental import pallas as pl
from jax.experimental.pallas import tpu as pltpu

B, SQ, H, D = 4, 32, 8, 128
SKV_SHARD = 4096
SPLIT = 4
SKV_LOCAL = SKV_SHARD // SPLIT
SCALE = D ** -0.5

PARTS = [(0, 4), (4, 7), (7, 8)]
ORDER = [(0, 1, 2), (1, 2, 0), (2, 0, 1)]
DRAIN = [(0, 0), (0, 1), (0, 2), (1, 0), (2, 0), (1, 1),
         (2, 1), (1, 2), (2, 2)]

SKIP_AR = os.environ.get("SKIP_AR") == "1"
SKIP_COMPUTE = os.environ.get("SKIP_COMPUTE") == "1"


def kernel(Q, K, V):
    def body(q_ref, k_hbm, v_hbm, out_ref,
             kv_buf, vv_buf, num_acc, den_acc, send16, recv16, recv_den,
             load_sems, send_sems, recv_sems, dsend_sems, drecv_sems):
        my_x = lax.axis_index("x")
        my_y = lax.axis_index("y")
        my_z = lax.axis_index("z")

        def nbr(axis):
            if axis == 0:
                return (1 - my_x, my_y, my_z)
            if axis == 1:
                return (my_x, 1 - my_y, my_z)
            return (my_x, my_y, 1 - my_z)

        neighbors = [nbr(0), nbr(1), nbr(2)]

        base = (2 * my_x + my_z) * SKV_LOCAL
        head_loads = []
        for h in range(H):
            per_head = []
            for b in range(B):
                per_head.append(pltpu.make_async_copy(
                    k_hbm.at[b, pl.ds(base, SKV_LOCAL), h, :],
                    kv_buf.at[h, b], load_sems.at[0],
                ))
                per_head.append(pltpu.make_async_copy(
                    v_hbm.at[b, pl.ds(base, SKV_LOCAL), h, :],
                    vv_buf.at[h, b], load_sems.at[1],
                ))
            head_loads.append(per_head)

        PD = 2
        for h in range(min(PD, H)):
            for ld in head_loads[h]:
                ld.start()

        barrier_sem = pltpu.get_barrier_semaphore()
        for n in neighbors:
            pl.semaphore_signal(
                barrier_sem, inc=1,
                device_id=n, device_id_type=pl.DeviceIdType.MESH,
            )
        pl.semaphore_wait(barrier_sem, 3)

        def compute_head(h):
            if h + PD < H:
                for ld in head_loads[h + PD]:
                    ld.start()
            for ld in head_loads[h]:
                ld.wait()
            q = q_ref[:, :, h, :].astype(jnp.bfloat16)
            k = kv_buf[h].astype(jnp.bfloat16)
            v = vv_buf[h].astype(jnp.bfloat16)
            s = lax.dot_general(
                q, k, (((2,), (2,)), ((0,), (0,))),
                preferred_element_type=jnp.float32,
            ) * SCALE
            e = jnp.exp(s)
            den_acc[h] = jnp.sum(e, axis=-1)
            num_acc[h] = lax.dot_general(
                e.astype(jnp.bfloat16), v, (((2,), (1,)), ((0,), (0,))),
                preferred_element_type=jnp.float32,
            )

        def start_stage(p, s):
            h0, h1 = PARTS[p]
            send16[h0:h1] = num_acc[h0:h1].astype(jnp.bfloat16)
            target = nbr(ORDER[p][s])
            r_num = pltpu.make_async_remote_copy(
                src_ref=send16.at[h0:h1],
                dst_ref=recv16.at[s, h0:h1],
                send_sem=send_sems.at[s, p],
                recv_sem=recv_sems.at[s, p],
                device_id=target, device_id_type=pl.DeviceIdType.MESH,
            )
            r_den = pltpu.make_async_remote_copy(
                src_ref=den_acc.at[h0:h1],
                dst_ref=recv_den.at[s, h0:h1],
                send_sem=dsend_sems.at[s, p],
                recv_sem=drecv_sems.at[s, p],
                device_id=target, device_id_type=pl.DeviceIdType.MESH,
            )
            r_num.start()
            r_den.start()
            return [r_num, r_den]

        def finish_stage(p, s, rdmas):
            h0, h1 = PARTS[p]
            for r in rdmas[(p, s)]:
                r.wait()
            num_acc[h0:h1] += recv16[s, h0:h1].astype(jnp.float32)
            den_acc[h0:h1] += recv_den[s, h0:h1]

        rdmas = {}
        if SKIP_COMPUTE:
            compute_head(0)
            if not SKIP_AR:
                for p in range(len(PARTS)):
                    rdmas[(p, 0)] = start_stage(p, 0)
        else:
            for p, (h0, h1) in enumerate(PARTS):
                for h in range(h0, h1):
                    compute_head(h)
                if not SKIP_AR:
                    rdmas[(p, 0)] = start_stage(p, 0)

        @functools.partial(
            pl.run_scoped, exit_sem=pltpu.SemaphoreType.REGULAR
        )
        def _(exit_sem):
            if not SKIP_AR:
                for p, s in DRAIN[:-1]:
                    finish_stage(p, s, rdmas)
                    if s < 2:
                        rdmas[(p, s + 1)] = start_stage(p, s + 1)
                    else:
                        h0, h1 = PARTS[p]
                        o = num_acc[h0:h1] / den_acc[h0:h1][..., None]
                        out_ref[:, :, h0:h1, :] = jnp.transpose(o, (1, 2, 0, 3))
                lp, ls = DRAIN[-1]
                finish_stage(lp, ls, rdmas)
            for n in neighbors:
                pl.semaphore_signal(
                    exit_sem, inc=1,
                    device_id=n, device_id_type=pl.DeviceIdType.MESH,
                )
            if not SKIP_AR:
                h0, h1 = PARTS[DRAIN[-1][0]]
                o = num_acc[h0:h1] / den_acc[h0:h1][..., None]
                out_ref[:, :, h0:h1, :] = jnp.transpose(o, (1, 2, 0, 3))
            else:
                o = num_acc[:] / den_acc[:][..., None]
                out_ref[:] = jnp.transpose(o, (1, 2, 0, 3))
            pl.semaphore_wait(exit_sem, 3)

    return pl.pallas_call(
        body,
        in_specs=[
            pl.BlockSpec(memory_space=pltpu.VMEM),
            pl.BlockSpec(memory_space=pl.ANY),
            pl.BlockSpec(memory_space=pl.ANY),
        ],
        out_specs=pl.BlockSpec(memory_space=pltpu.VMEM),
        out_shape=jax.ShapeDtypeStruct((B, SQ, H, D), jnp.float32),
        scratch_shapes=[
            pltpu.VMEM((H, B, SKV_LOCAL, D), jnp.float32),
            pltpu.VMEM((H, B, SKV_LOCAL, D), jnp.float32),
            pltpu.VMEM((H, B, SQ, D), jnp.float32),
            pltpu.VMEM((H, B, SQ), jnp.float32),
            pltpu.VMEM((H, B, SQ, D), jnp.bfloat16),
            pltpu.VMEM((3, H, B, SQ, D), jnp.bfloat16),
            pltpu.VMEM((3, H, B, SQ), jnp.float32),
            pltpu.SemaphoreType.DMA((2,)),
            pltpu.SemaphoreType.DMA((3, 3)),
            pltpu.SemaphoreType.DMA((3, 3)),
            pltpu.SemaphoreType.DMA((3, 3)),
            pltpu.SemaphoreType.DMA((3, 3)),
        ],
        compiler_params=pltpu.CompilerParams(
            collective_id=0, vmem_limit_bytes=64 * 1024 * 1024,
        ),
    )(Q, K, V)


# device time: 28151 ns/iter; 3.9114x vs baseline; 1.1572x over previous
import functools
import os

import jax
import jax.numpy as jnp
from jax import lax
from jax.experimental import pallas as pl
from jax.experimental.pallas import tpu as pltpu

B, SQ, H, D = 4, 32, 8, 128
SKV_SHARD = 4096
SPLIT = 4
SKV_LOCAL = SKV_SHARD // SPLIT
SCALE = D ** -0.5

PARTS = [(0, 3), (3, 6), (6, 8)]
ORDER = [(0, 1, 2), (1, 2, 0), (2, 0, 1)]
DRAIN = [(0, 0), (1, 0), (0, 1), (2, 0), (1, 1), (0, 2),
         (2, 1), (1, 2), (2, 2)]

SKIP_AR = os.environ.get("SKIP_AR") == "1"
SKIP_COMPUTE = os.environ.get("SKIP_COMPUTE") == "1"


def kernel(Q, K, V):
    def body(q_ref, k_hbm, v_hbm, out_ref,
             kv_buf, vv_buf, num_acc, den_acc, send16, recv16, recv_den,
             load_sems, send_sems, recv_sems, dsend_sems, drecv_sems):
        my_x = lax.axis_index("x")
        my_y = lax.axis_index("y")
        my_z = lax.axis_index("z")

        def nbr(axis):
            if axis == 0:
                return (1 - my_x, my_y, my_z)
            if axis == 1:
                return (my_x, 1 - my_y, my_z)
            return (my_x, my_y, 1 - my_z)

        neighbors = [nbr(0), nbr(1), nbr(2)]

        base = (2 * my_x + my_z) * SKV_LOCAL
        head_loads = []
        for h in range(H):
            per_head = []
            for b in range(B):
                per_head.append(pltpu.make_async_copy(
                    k_hbm.at[b, pl.ds(base, SKV_LOCAL), h, :],
                    kv_buf.at[h, b], load_sems.at[0],
                ))
                per_head.append(pltpu.make_async_copy(
                    v_hbm.at[b, pl.ds(base, SKV_LOCAL), h, :],
                    vv_buf.at[h, b], load_sems.at[1],
                ))
            head_loads.append(per_head)

        PD = 2
        for h in range(min(PD, H)):
            for ld in head_loads[h]:
                ld.start()

        barrier_sem = pltpu.get_barrier_semaphore()
        for n in neighbors:
            pl.semaphore_signal(
                barrier_sem, inc=1,
                device_id=n, device_id_type=pl.DeviceIdType.MESH,
            )
        pl.semaphore_wait(barrier_sem, 3)

        def compute_head(h):
            if h + PD < H:
                for ld in head_loads[h + PD]:
                    ld.start()
            for ld in head_loads[h]:
                ld.wait()
            q = q_ref[:, :, h, :].astype(jnp.bfloat16)
            k = kv_buf[h].astype(jnp.bfloat16)
            v = vv_buf[h].astype(jnp.bfloat16)
            s = lax.dot_general(
                q, k, (((2,), (2,)), ((0,), (0,))),
                preferred_element_type=jnp.float32,
            ) * SCALE
            e = jnp.exp(s)
            den_acc[h] = jnp.sum(e, axis=-1)
            num_acc[h] = lax.dot_general(
                e.astype(jnp.bfloat16), v, (((2,), (1,)), ((0,), (0,))),
                preferred_element_type=jnp.float32,
            )

        def start_stage(p, s):
            h0, h1 = PARTS[p]
            send16[h0:h1] = num_acc[h0:h1].astype(jnp.bfloat16)
            target = nbr(ORDER[p][s])
            r_num = pltpu.make_async_remote_copy(
                src_ref=send16.at[h0:h1],
                dst_ref=recv16.at[s, h0:h1],
                send_sem=send_sems.at[s, p],
                recv_sem=recv_sems.at[s, p],
                device_id=target, device_id_type=pl.DeviceIdType.MESH,
            )
            r_den = pltpu.make_async_remote_copy(
                src_ref=den_acc.at[h0:h1],
                dst_ref=recv_den.at[s, h0:h1],
                send_sem=dsend_sems.at[s, p],
                recv_sem=drecv_sems.at[s, p],
                device_id=target, device_id_type=pl.DeviceIdType.MESH,
            )
            r_num.start()
            r_den.start()
            return [r_num, r_den]

        def finish_stage(p, s, rdmas):
            h0, h1 = PARTS[p]
            for r in rdmas[(p, s)]:
                r.wait()
            num_acc[h0:h1] += recv16[s, h0:h1].astype(jnp.float32)
            den_acc[h0:h1] += recv_den[s, h0:h1]

        rdmas = {}
        if SKIP_COMPUTE:
            compute_head(0)
            if not SKIP_AR:
                for p in range(len(PARTS)):
                    rdmas[(p, 0)] = start_stage(p, 0)
        else:
            for p, (h0, h1) in enumerate(PARTS):
                for h in range(h0, h1):
                    compute_head(h)
                if not SKIP_AR:
                    rdmas[(p, 0)] = start_stage(p, 0)

        @functools.partial(
            pl.run_scoped, exit_sem=pltpu.SemaphoreType.REGULAR
        )
        def _(exit_sem):
            if not SKIP_AR:
                for p, s in DRAIN[:-1]:
                    finish_stage(p, s, rdmas)
                    if s < 2:
                        rdmas[(p, s + 1)] = start_stage(p, s + 1)
                    else:
                        h0, h1 = PARTS[p]
                        o = num_acc[h0:h1] / den_acc[h0:h1][..., None]
                        out_ref[:, :, h0:h1, :] = jnp.transpose(o, (1, 2, 0, 3))
                lp, ls = DRAIN[-1]
                finish_stage(lp, ls, rdmas)
            for n in neighbors:
                pl.semaphore_signal(
                    exit_sem, inc=1,
                    device_id=n, device_id_type=pl.DeviceIdType.MESH,
                )
            if not SKIP_AR:
                h0, h1 = PARTS[DRAIN[-1][0]]
                o = num_acc[h0:h1] / den_acc[h0:h1][..., None]
                out_ref[:, :, h0:h1, :] = jnp.transpose(o, (1, 2, 0, 3))
            else:
                o = num_acc[:] / den_acc[:][..., None]
                out_ref[:] = jnp.transpose(o, (1, 2, 0, 3))
            pl.semaphore_wait(exit_sem, 3)

    return pl.pallas_call(
        body,
        in_specs=[
            pl.BlockSpec(memory_space=pltpu.VMEM),
            pl.BlockSpec(memory_space=pl.ANY),
            pl.BlockSpec(memory_space=pl.ANY),
        ],
        out_specs=pl.BlockSpec(memory_space=pltpu.VMEM),
        out_shape=jax.ShapeDtypeStruct((B, SQ, H, D), jnp.float32),
        scratch_shapes=[
            pltpu.VMEM((H, B, SKV_LOCAL, D), jnp.float32),
            pltpu.VMEM((H, B, SKV_LOCAL, D), jnp.float32),
            pltpu.VMEM((H, B, SQ, D), jnp.float32),
            pltpu.VMEM((H, B, SQ), jnp.float32),
            pltpu.VMEM((H, B, SQ, D), jnp.bfloat16),
            pltpu.VMEM((3, H, B, SQ, D), jnp.bfloat16),
            pltpu.VMEM((3, H, B, SQ), jnp.float32),
            pltpu.SemaphoreType.DMA((2,)),
            pltpu.SemaphoreType.DMA((3, 3)),
            pltpu.SemaphoreType.DMA((3, 3)),
            pltpu.SemaphoreType.DMA((3, 3)),
            pltpu.SemaphoreType.DMA((3, 3)),
        ],
        compiler_params=pltpu.CompilerParams(
            collective_id=0, vmem_limit_bytes=64 * 1024 * 1024,
        ),
    )(Q, K, V)
